# baseline (speedup 1.0000x reference)
import functools

import jax
import jax.numpy as jnp
from jax import lax
from jax.experimental import pallas as pl
from jax.experimental.pallas import tpu as pltpu
from jax.experimental.pallas import tpu_sc as plsc

N = 10000
E = 320000
F = 128
H1 = 128
H2 = 512
C = 4
G = 16
K = 5

NC = 2
NS = 16
NW = NC * NS
RPW = 4
NR = NW * RPW
RPG = 80
NPAD = NR * RPG
CAP = 4096


def _edge_layout(src, dst):
    deg = jnp.zeros((N,), jnp.float32).at[src].add(1.0)
    dinv = jnp.where(deg > 0, lax.rsqrt(jnp.maximum(deg, 1e-9)), 0.0)
    norm = -(dinv[src] * dinv[dst])
    order = jnp.argsort(dst)
    src_s = src[order]
    dst_s = dst[order]
    norm_s = norm[order]
    rid = dst_s // RPG
    starts = jnp.searchsorted(dst_s, (jnp.arange(NR) * RPG).astype(dst_s.dtype)).astype(jnp.int32)
    pos = jnp.arange(E, dtype=jnp.int32) - starts[rid]
    ok = pos < CAP
    slot = jnp.where(ok, rid * CAP + pos, NR * CAP)
    srcp = jnp.zeros((NR * CAP,), jnp.int32).at[slot].set(src_s, mode="drop")
    dstlp = jnp.zeros((NR * CAP,), jnp.int32).at[slot].set(
        (dst_s - rid * RPG).astype(jnp.int32), mode="drop")
    normp = jnp.zeros((NR * CAP,), jnp.float32).at[slot].set(norm_s, mode="drop")
    ends = jnp.concatenate([starts[1:], jnp.array([E], jnp.int32)])
    counts = jnp.minimum(ends - starts, CAP)
    return srcp, dstlp, normp, counts


_SPMM_CACHE = {}


def _make_spmm(Fw, with_prev):
    key = (Fw, with_prev)
    if key in _SPMM_CACHE:
        return _SPMM_CACHE[key]
    CHUNK = 64 if Fw <= 128 else 16
    NCOL = Fw // 16
    mesh = plsc.VectorSubcoreMesh(core_axis_name="c", subcore_axis_name="s")

    scratch = [
        pltpu.VMEM((CAP,), jnp.int32),
        pltpu.VMEM((CAP,), jnp.int32),
        pltpu.VMEM((CAP,), jnp.float32),
        pltpu.VMEM((NR,), jnp.int32),
        pltpu.VMEM((RPG * Fw,), jnp.float32),
        pltpu.VMEM((CHUNK, Fw), jnp.float32),
        pltpu.VMEM((CHUNK, Fw), jnp.float32),
        pltpu.VMEM((16, Fw), jnp.float32),
        pltpu.SemaphoreType.DMA,
        pltpu.SemaphoreType.DMA,
    ]

    def body(v_hbm, srcp_hbm, dstlp_hbm, normp_hbm, cnts_hbm, *rest):
        if with_prev:
            (prev_hbm, out_hbm, src_v, dstl_v, norm_v, cnt_v, acc,
             rows0, rows1, ptile, sem0, sem1) = rest
        else:
            (out_hbm, src_v, dstl_v, norm_v, cnt_v, acc,
             rows0, rows1, ptile, sem0, sem1) = rest
        wid = lax.axis_index("s") * NC + lax.axis_index("c")
        pltpu.sync_copy(cnts_hbm, cnt_v)
        iota16 = lax.iota(jnp.int32, 16)

        def start(g, buf, sem):
            off = pl.multiple_of(g * CHUNK, CHUNK)
            pltpu.async_copy(v_hbm.at[src_v.at[pl.ds(off, CHUNK)]], buf, sem)

        def wait(buf, sem):
            pltpu.make_async_copy(
                v_hbm.at[src_v.at[pl.ds(0, CHUNK)]], buf, sem).wait()

        def process(buf, gbase):
            for e in range(CHUNK):
                sp = jnp.full((16,), gbase + e, jnp.int32)
                nrm = plsc.load_gather(norm_v, [sp])
                dr = plsc.load_gather(dstl_v, [sp])
                base = dr * Fw + iota16
                for c in range(NCOL):
                    val = buf[e, pl.ds(c * 16, 16)] * nrm
                    plsc.addupdate_scatter(acc, [base + (c * 16)], val)

        def range_body(r, carry):
            rid = wid * RPW + r
            ebase = rid * CAP
            pltpu.sync_copy(srcp_hbm.at[pl.ds(ebase, CAP)], src_v)
            pltpu.sync_copy(dstlp_hbm.at[pl.ds(ebase, CAP)], dstl_v)
            pltpu.sync_copy(normp_hbm.at[pl.ds(ebase, CAP)], norm_v)

            def zero_body(i, c2):
                acc[pl.ds(i * 16, 16)] = jnp.zeros((16,), jnp.float32)
                return c2
            lax.fori_loop(0, RPG * Fw // 16, zero_body, 0)

            cnt = jnp.max(plsc.load_gather(cnt_v, [jnp.full((16,), rid, jnp.int32)]))
            npair = jnp.maximum((cnt + (2 * CHUNK - 1)) // (2 * CHUNK), 1)

            start(0, rows0, sem0)

            def pair_body(p, c2):
                start(2 * p + 1, rows1, sem1)
                wait(rows0, sem0)
                process(rows0, 2 * p * CHUNK)

                @pl.when(p + 1 < npair)
                def _():
                    start(2 * p + 2, rows0, sem0)
                wait(rows1, sem1)
                process(rows1, (2 * p + 1) * CHUNK)
                return c2
            lax.fori_loop(0, npair, pair_body, 0)

            rowbase = rid * RPG
            if with_prev:
                def tile_body(t, c2):
                    row0 = rowbase + t * 16
                    pltpu.sync_copy(prev_hbm.at[pl.ds(row0, 16)], ptile)

                    def row_body(j, c3):
                        for c in range(NCOL):
                            a = acc[pl.ds((t * 16 + j) * Fw + c * 16, 16)]
                            ptile[j, pl.ds(c * 16, 16)] = (
                                2.0 * a - ptile[j, pl.ds(c * 16, 16)])
                        return c3
                    lax.fori_loop(0, 16, row_body, 0)
                    pltpu.sync_copy(ptile, out_hbm.at[pl.ds(row0, 16)])
                    return c2
                lax.fori_loop(0, RPG // 16, tile_body, 0)
            else:
                def tile_body(t, c2):
                    row0 = rowbase + t * 16

                    def row_body(j, c3):
                        for c in range(NCOL):
                            ptile[j, pl.ds(c * 16, 16)] = acc[
                                pl.ds((t * 16 + j) * Fw + c * 16, 16)]
                        return c3
                    lax.fori_loop(0, 16, row_body, 0)
                    pltpu.sync_copy(ptile, out_hbm.at[pl.ds(row0, 16)])
                    return c2
                lax.fori_loop(0, RPG // 16, tile_body, 0)
            return carry
        lax.fori_loop(0, RPW, range_body, 0)

    fn = pl.kernel(
        body,
        out_type=jax.ShapeDtypeStruct((NPAD, Fw), jnp.float32),
        mesh=mesh,
        scratch_types=scratch,
        compiler_params=pltpu.CompilerParams(needs_layout_passes=False),
    )
    _SPMM_CACHE[key] = fn
    return fn


_BM = 1024


def _matmul_body(relu, gate, x_ref, w_ref, b_ref, *rest):
    if gate:
        wg_ref, bg_ref, o_ref, o2_ref, acc_ref = rest
    else:
        o_ref, acc_ref = rest
    k = pl.program_id(1)

    @pl.when(k == 0)
    def _():
        acc_ref[...] = jnp.zeros_like(acc_ref)
    acc_ref[...] += jnp.dot(x_ref[...], w_ref[...],
                            preferred_element_type=jnp.float32)

    @pl.when(k == pl.num_programs(1) - 1)
    def _():
        r = acc_ref[...] + b_ref[...]
        if relu:
            r = jnp.maximum(r, 0.0)
        o_ref[...] = r
        if gate:
            o2_ref[...] = jnp.dot(r, wg_ref[...],
                                  preferred_element_type=jnp.float32) + bg_ref[...]


def _matmul(X, Wm, bias, relu, gate_w=None, gate_b=None):
    M, Kd = X.shape
    Nout = Wm.shape[1]
    KB = Kd // 128
    gate = gate_w is not None
    in_specs = [
        pl.BlockSpec((_BM, 128), lambda i, k: (i, k)),
        pl.BlockSpec((128, Nout), lambda i, k: (k, 0)),
        pl.BlockSpec((1, Nout), lambda i, k: (0, 0)),
    ]
    out_specs = pl.BlockSpec((_BM, Nout), lambda i, k: (i, 0))
    out_shape = jax.ShapeDtypeStruct((M, Nout), jnp.float32)
    if gate:
        in_specs += [
            pl.BlockSpec((Nout, 128), lambda i, k: (0, 0)),
            pl.BlockSpec((1, 128), lambda i, k: (0, 0)),
        ]
        out_specs = [out_specs, pl.BlockSpec((_BM, 128), lambda i, k: (i, 0))]
        out_shape = [out_shape, jax.ShapeDtypeStruct((M, 128), jnp.float32)]
    args = (X, Wm, bias) + ((gate_w, gate_b) if gate else ())
    return pl.pallas_call(
        functools.partial(_matmul_body, relu, gate),
        grid=(M // _BM, KB),
        in_specs=in_specs,
        out_specs=out_specs,
        out_shape=out_shape,
        scratch_shapes=[pltpu.VMEM((_BM, Nout), jnp.float32)],
    )(*args)


def _wts_body(gate_ref, batch_ref, wts_ref):
    g = gate_ref[...]
    b = batch_ref[...]
    w = jnp.zeros_like(g)
    for seg in range(G):
        m = b == seg
        mg = jnp.max(jnp.where(m, g, -jnp.inf))
        e = jnp.exp(g - mg)
        s = jnp.sum(jnp.where(m, e, 0.0))
        w = jnp.where(m, e / s, w)
    wts_ref[...] = w


def _pool_body(h_ref, wts_ref, batch_ref, wfc_ref, bfc_ref, out_ref, pooled_ref):
    i = pl.program_id(0)

    @pl.when(i == 0)
    def _():
        pooled_ref[...] = jnp.zeros_like(pooled_ref)
    eq = lax.broadcasted_iota(jnp.int32, (G, _BM), 0) == batch_ref[0]
    oh = jnp.where(eq, wts_ref[0], jnp.zeros((1, 1), jnp.float32))
    pooled_ref[...] += jnp.dot(oh, h_ref[...], preferred_element_type=jnp.float32)

    @pl.when(i == pl.num_programs(0) - 1)
    def _():
        logits = jnp.dot(pooled_ref[...], wfc_ref[...],
                         preferred_element_type=jnp.float32) + bfc_ref[...]
        colmask = lax.broadcasted_iota(jnp.int32, (G, 128), 1) < C
        neg = jnp.where(colmask, logits, -jnp.inf)
        mx = jnp.max(neg, axis=1, keepdims=True)
        lse = jnp.log(jnp.sum(jnp.where(colmask, jnp.exp(neg - mx), 0.0),
                              axis=1, keepdims=True)) + mx
        out_ref[...] = logits - lse


def _basis(v, L, Fw):
    spmm0 = _make_spmm(Fw, False)
    spmm1 = _make_spmm(Fw, True)
    srcp, dstlp, normp, counts = L
    t1 = spmm0(v, srcp, dstlp, normp, counts)
    t2 = spmm1(t1, srcp, dstlp, normp, counts, v)
    t3 = spmm1(t2, srcp, dstlp, normp, counts, t1)
    t4 = spmm1(t3, srcp, dstlp, normp, counts, t2)
    return [t1, t2, t3, t4]


def _bidir_block(v, LF, LB, Wf, bf, Wb, bb, extra_x=None, extra_W=None,
                 extra_b=None, relu=True, gate_w=None, gate_b=None):
    Fw = v.shape[1]
    Tf = _basis(v, LF, Fw)
    Tb = _basis(v, LB, Fw)
    Xs = [v] + Tf + Tb
    Ws = [Wf[0] + Wb[0]] + [Wf[k] for k in range(1, K)] + [Wb[k] for k in range(1, K)]
    bias = bf + bb
    if extra_x is not None:
        Xs.append(extra_x)
        Ws.append(extra_W)
        bias = bias + extra_b
    X = jnp.concatenate(Xs, axis=1)
    Wm = jnp.concatenate(Ws, axis=0)
    return _matmul(X, Wm, bias[None, :], relu, gate_w, gate_b)


def kernel(x, edge_index, batch, Wbf1, bbf1, Wbb1, bbb1, Wraf, braf, Wrab,
           brab, Wrbf, brbf, Wrbb, brbb, Wres, bres, Wg, bg, Wfc, bfc):
    src, dst = edge_index[0], edge_index[1]
    LF = _edge_layout(src, dst)
    LB = _edge_layout(dst, src)
    xp = jnp.zeros((NPAD, F), jnp.float32).at[:N].set(x)

    h = _bidir_block(xp, LF, LB, Wbf1, bbf1, Wbb1, bbb1)
    t = _bidir_block(h, LF, LB, Wraf, braf, Wrab, brab)
    wg_pad = jnp.zeros((H2, 128), jnp.float32).at[:, :1].set(Wg)
    bg_pad = jnp.zeros((1, 128), jnp.float32).at[0, :1].set(bg)
    h2, gbuf = _bidir_block(t, LF, LB, Wrbf, brbf, Wrbb, brbb,
                            extra_x=h, extra_W=Wres, extra_b=bres,
                            relu=True, gate_w=wg_pad, gate_b=bg_pad)

    batch_pad = jnp.concatenate([batch, jnp.full((NPAD - N,), G, jnp.int32)])
    gate2d = gbuf[:, 0].reshape(NPAD // 128, 128)
    wts2d = pl.pallas_call(
        _wts_body,
        out_shape=jax.ShapeDtypeStruct((NPAD // 128, 128), jnp.float32),
    )(gate2d, batch_pad.reshape(NPAD // 128, 128))

    wfc_pad = jnp.zeros((H2, 128), jnp.float32).at[:, :C].set(Wfc)
    bfc_pad = jnp.zeros((1, 128), jnp.float32).at[0, :C].set(bfc)
    NB = NPAD // _BM
    out128 = pl.pallas_call(
        _pool_body,
        grid=(NB,),
        in_specs=[
            pl.BlockSpec((_BM, H2), lambda i: (i, 0)),
            pl.BlockSpec((1, 1, _BM), lambda i: (i, 0, 0)),
            pl.BlockSpec((1, 1, _BM), lambda i: (i, 0, 0)),
            pl.BlockSpec((H2, 128), lambda i: (0, 0)),
            pl.BlockSpec((1, 128), lambda i: (0, 0)),
        ],
        out_specs=pl.BlockSpec((G, 128), lambda i: (0, 0)),
        out_shape=jax.ShapeDtypeStruct((G, 128), jnp.float32),
        scratch_shapes=[pltpu.VMEM((G, H2), jnp.float32)],
    )(h2, wts2d.reshape(NB, 1, _BM), batch_pad.reshape(NB, 1, _BM),
      wfc_pad, bfc_pad)
    return out128[:G, :C]
